# Initial kernel scaffold; baseline (speedup 1.0000x reference)
import jax
import jax.numpy as jnp
from jax.experimental import pallas as pl


def kernel(tensor, table):
    raise NotImplementedError("write your pallas kernel here")


# speedup vs baseline: 1.0602x; 1.0602x over previous
import dataclasses

import jax
import jax.numpy as jnp
from jax import lax
from jax.experimental import pallas as pl
from jax.experimental.pallas import tpu as pltpu
from jax.experimental.pallas import tpu_sc as plsc

_NC, _NS = 2, 16
_NW = _NC * _NS
_CHUNK = 256


def kernel(tensor, table):
    batch, hist = tensor.shape
    vocab, embed = table.shape
    n = batch * hist
    per_w = n // _NW
    width = 2 * embed
    idx_flat = tensor.reshape(n)
    table_pairs = table.reshape(vocab // 2, width)

    mesh = plsc.VectorSubcoreMesh(
        core_axis_name="core", subcore_axis_name="subcore"
    )
    cp = pltpu.CompilerParams()
    if "needs_layout_passes" in pltpu.CompilerParams.__dataclass_fields__:
        cp = dataclasses.replace(cp, needs_layout_passes=False)

    @pl.kernel(
        compiler_params=cp,
        out_type=jax.ShapeDtypeStruct((n, embed), table.dtype),
        mesh=mesh,
        scratch_types=[
            pltpu.VMEM((_CHUNK,), jnp.int32),
            pltpu.VMEM((_CHUNK,), jnp.int32),
            pltpu.VMEM((_CHUNK,), jnp.int32),
            pltpu.VMEM((_CHUNK, width), jnp.float32),
            pltpu.VMEM((_CHUNK, embed), jnp.float32),
            pltpu.SemaphoreType.DMA,
        ],
    )
    def gather_kernel(
        table_hbm, idx_hbm, out_hbm, idx_raw, idx_half, par64, rows_v, out_v, sem
    ):
        wid = lax.axis_index("subcore") * _NC + lax.axis_index("core")
        base = wid * per_w
        lanes = lax.iota(jnp.int32, 16)

        @pl.loop(0, per_w, step=_CHUNK)
        def _(off):
            pltpu.sync_copy(idx_hbm.at[pl.ds(base + off, _CHUNK)], idx_raw)

            @pl.loop(0, _CHUNK, step=16)
            def _(i):
                v = idx_raw[pl.ds(i, 16)]
                idx_half[pl.ds(i, 16)] = lax.shift_right_logical(v, 1)
                par64[pl.ds(i, 16)] = lax.shift_left(lax.bitwise_and(v, 1), 6)

            pltpu.async_copy(table_hbm.at[idx_half], rows_v, sem).wait()

            @pl.loop(0, _CHUNK)
            def _(r):
                rsplat = jnp.zeros((16,), jnp.int32) + r
                p = plsc.load_gather(par64, [rsplat])
                for c in range(embed // 16):
                    col = p + (c * 16 + lanes)
                    x = plsc.load_gather(rows_v, [rsplat, col])
                    out_v[r, pl.ds(c * 16, 16)] = x

            pltpu.sync_copy(out_v, out_hbm.at[pl.ds(base + off, _CHUNK)])

    out = gather_kernel(table_pairs, idx_flat)
    return out.reshape(batch, hist, embed)


# speedup vs baseline: 1.3217x; 1.2467x over previous
import dataclasses

import jax
import jax.numpy as jnp
from jax import lax
from jax.experimental import pallas as pl
from jax.experimental.pallas import tpu as pltpu
from jax.experimental.pallas import tpu_sc as plsc

_NC, _NS = 2, 16
_NW = _NC * _NS
_CHUNK = 256


def kernel(tensor, table):
    batch, hist = tensor.shape
    vocab, embed = table.shape
    n = batch * hist
    per_w = n // _NW
    nchunk = per_w // _CHUNK
    width = 2 * embed
    idx_flat = tensor.reshape(n)
    table_pairs = table.reshape(vocab // 2, width)

    mesh = plsc.VectorSubcoreMesh(
        core_axis_name="core", subcore_axis_name="subcore"
    )
    cp = pltpu.CompilerParams()
    if "needs_layout_passes" in pltpu.CompilerParams.__dataclass_fields__:
        cp = dataclasses.replace(cp, needs_layout_passes=False)

    @pl.kernel(
        compiler_params=cp,
        out_type=jax.ShapeDtypeStruct((n // 2, width), table.dtype),
        mesh=mesh,
        scratch_types=[
            pltpu.VMEM((per_w,), jnp.int32),
            pltpu.VMEM((_CHUNK,), jnp.int32),
            pltpu.VMEM((_CHUNK,), jnp.int32),
            pltpu.VMEM((_CHUNK,), jnp.int32),
            pltpu.VMEM((_CHUNK,), jnp.int32),
            pltpu.VMEM((_CHUNK, width), jnp.float32),
            pltpu.VMEM((_CHUNK, width), jnp.float32),
            pltpu.VMEM((_CHUNK // 2, width), jnp.float32),
            pltpu.VMEM((_CHUNK // 2, width), jnp.float32),
            pltpu.SemaphoreType.DMA,
            pltpu.SemaphoreType.DMA,
            pltpu.SemaphoreType.DMA,
            pltpu.SemaphoreType.DMA,
        ],
    )
    def gather_kernel(
        table_hbm, idx_hbm, out_hbm,
        idx_all, half0, half1, par0, par1, rows0, rows1, out0, out1,
        gsem0, gsem1, osem0, osem1,
    ):
        wid = lax.axis_index("subcore") * _NC + lax.axis_index("core")
        base = wid * per_w
        lanes = lax.iota(jnp.int32, 16)

        pltpu.sync_copy(idx_hbm.at[pl.ds(base, per_w)], idx_all)

        def prep(off, half, par):
            @pl.loop(0, _CHUNK, step=16)
            def _(i):
                v = idx_all[pl.ds(off + i, 16)]
                half[pl.ds(i, 16)] = lax.shift_right_logical(v, 1)
                par[pl.ds(i, 16)] = lax.shift_left(lax.bitwise_and(v, 1), 6)

        def start_gather(half, rows, gsem):
            return pltpu.async_copy(table_hbm.at[half], rows, gsem)

        def select(rows, par, out):
            @pl.loop(0, _CHUNK, step=2)
            def _(r):
                j = lax.shift_right_logical(r, 1)
                r0 = jnp.zeros((16,), jnp.int32) + r
                r1 = r0 + 1
                p0 = plsc.load_gather(par, [r0])
                p1 = plsc.load_gather(par, [r1])
                for c in range(embed // 16):
                    x = plsc.load_gather(rows, [r0, p0 + (c * 16 + lanes)])
                    out[j, pl.ds(c * 16, 16)] = x
                for c in range(embed // 16):
                    x = plsc.load_gather(rows, [r1, p1 + (c * 16 + lanes)])
                    out[j, pl.ds(embed + c * 16, 16)] = x

        def start_out(out, off, osem):
            row = pl.multiple_of((base + off) // 2, _CHUNK // 2)
            return pltpu.async_copy(
                out, out_hbm.at[pl.ds(row, _CHUNK // 2)], osem
            )

        prep(0, half0, par0)
        g0 = start_gather(half0, rows0, gsem0)
        prep(_CHUNK, half1, par1)
        g1 = start_gather(half1, rows1, gsem1)

        @pl.loop(0, (nchunk - 2) // 2)
        def _(k):
            off = 2 * k * _CHUNK
            g0.wait()
            select(rows0, par0, out0)
            o0 = start_out(out0, off, osem0)
            g1.wait()
            select(rows1, par1, out1)
            o1 = start_out(out1, off + _CHUNK, osem1)
            o0.wait()
            prep(off + 2 * _CHUNK, half0, par0)
            start_gather(half0, rows0, gsem0)
            o1.wait()
            prep(off + 3 * _CHUNK, half1, par1)
            start_gather(half1, rows1, gsem1)

        last = (nchunk - 2) * _CHUNK
        g0.wait()
        select(rows0, par0, out0)
        o0 = start_out(out0, last, osem0)
        g1.wait()
        select(rows1, par1, out1)
        o1 = start_out(out1, last + _CHUNK, osem1)
        o0.wait()
        o1.wait()

    out = gather_kernel(table_pairs, idx_flat)
    return out.reshape(batch, hist, embed)


# speedup vs baseline: 1.5046x; 1.1384x over previous
import dataclasses

import jax
import jax.numpy as jnp
from jax import lax
from jax.experimental import pallas as pl
from jax.experimental.pallas import tpu as pltpu
from jax.experimental.pallas import tpu_sc as plsc

_NC, _NS = 2, 16
_NW = _NC * _NS
_BLK = 4


def kernel(tensor, table):
    batch, hist = tensor.shape
    vocab, embed = table.shape
    n = batch * hist
    width = 2 * embed
    per_wb = batch // _NW
    per_w = per_wb * hist
    rows_blk = _BLK * hist
    nchunk = per_wb // _BLK
    idx_flat = tensor.reshape(n)
    table_pairs = table.reshape(vocab // 2, width)

    mesh = plsc.VectorSubcoreMesh(
        core_axis_name="core", subcore_axis_name="subcore"
    )
    cp = pltpu.CompilerParams()
    if "needs_layout_passes" in pltpu.CompilerParams.__dataclass_fields__:
        cp = dataclasses.replace(cp, needs_layout_passes=False)

    @pl.kernel(
        compiler_params=cp,
        out_type=jax.ShapeDtypeStruct((batch, hist, embed), table.dtype),
        mesh=mesh,
        scratch_types=[
            pltpu.VMEM((per_w,), jnp.int32),
            pltpu.VMEM((rows_blk,), jnp.int32),
            pltpu.VMEM((rows_blk,), jnp.int32),
            pltpu.VMEM((rows_blk,), jnp.int32),
            pltpu.VMEM((rows_blk,), jnp.int32),
            pltpu.VMEM((rows_blk, width), jnp.float32),
            pltpu.VMEM((rows_blk, width), jnp.float32),
            pltpu.VMEM((rows_blk, embed), jnp.float32),
            pltpu.VMEM((rows_blk, embed), jnp.float32),
            pltpu.SemaphoreType.DMA,
            pltpu.SemaphoreType.DMA,
            pltpu.SemaphoreType.DMA,
            pltpu.SemaphoreType.DMA,
        ],
    )
    def gather_kernel(
        table_hbm, idx_hbm, out_hbm,
        idx_all, half0, half1, par0, par1, rows0, rows1, out0, out1,
        gsem0, gsem1, osem0, osem1,
    ):
        wid = lax.axis_index("subcore") * _NC + lax.axis_index("core")
        base = wid * per_w
        bbase = wid * per_wb
        lanes = lax.iota(jnp.int32, 16)

        pltpu.sync_copy(idx_hbm.at[pl.ds(base, per_w)], idx_all)

        def prep(off, half, par):
            @pl.loop(0, rows_blk, step=16)
            def _(i):
                v = idx_all[pl.ds(off + i, 16)]
                half[pl.ds(i, 16)] = lax.shift_right_logical(v, 1)
                par[pl.ds(i, 16)] = lax.shift_left(lax.bitwise_and(v, 1), 6)

        def start_gather(half, rows, gsem):
            return pltpu.async_copy(table_hbm.at[half], rows, gsem)

        def select(rows, par, out):
            @pl.loop(0, rows_blk, step=2)
            def _(r):
                r0 = jnp.zeros((16,), jnp.int32) + r
                r1 = r0 + 1
                p0 = plsc.load_gather(par, [r0])
                p1 = plsc.load_gather(par, [r1])
                for c in range(embed // 16):
                    x = plsc.load_gather(rows, [r0, p0 + (c * 16 + lanes)])
                    out[r, pl.ds(c * 16, 16)] = x
                for c in range(embed // 16):
                    x = plsc.load_gather(rows, [r1, p1 + (c * 16 + lanes)])
                    out[r + 1, pl.ds(c * 16, 16)] = x

        def start_out(out, chunk, osem):
            return pltpu.async_copy(
                out.reshape(_BLK, hist, embed),
                out_hbm.at[pl.ds(bbase + chunk * _BLK, _BLK)],
                osem,
            )

        prep(0, half0, par0)
        g0 = start_gather(half0, rows0, gsem0)
        prep(rows_blk, half1, par1)
        g1 = start_gather(half1, rows1, gsem1)

        @pl.loop(0, (nchunk - 2) // 2)
        def _(k):
            off = 2 * k * rows_blk
            g0.wait()
            select(rows0, par0, out0)
            o0 = start_out(out0, 2 * k, osem0)
            g1.wait()
            select(rows1, par1, out1)
            o1 = start_out(out1, 2 * k + 1, osem1)
            o0.wait()
            prep(off + 2 * rows_blk, half0, par0)
            start_gather(half0, rows0, gsem0)
            o1.wait()
            prep(off + 3 * rows_blk, half1, par1)
            start_gather(half1, rows1, gsem1)

        g0.wait()
        select(rows0, par0, out0)
        o0 = start_out(out0, nchunk - 2, osem0)
        g1.wait()
        select(rows1, par1, out1)
        o1 = start_out(out1, nchunk - 1, osem1)
        o0.wait()
        o1.wait()

    return gather_kernel(table_pairs, idx_flat)


# speedup vs baseline: 1.9387x; 1.2885x over previous
import dataclasses

import jax
import jax.numpy as jnp
from jax import lax
from jax.experimental import pallas as pl
from jax.experimental.pallas import tpu as pltpu
from jax.experimental.pallas import tpu_sc as plsc

_NC, _NS = 2, 16
_NW = _NC * _NS
_BLK = 4


def kernel(tensor, table):
    batch, hist = tensor.shape
    vocab, embed = table.shape
    n = batch * hist
    width = 2 * embed
    per_wb = batch // _NW
    per_w = per_wb * hist
    rows_blk = _BLK * hist
    nchunk = per_wb // _BLK
    idx_flat = tensor.reshape(n)
    table_pairs = table.reshape(vocab // 2, width)

    mesh = plsc.VectorSubcoreMesh(
        core_axis_name="core", subcore_axis_name="subcore"
    )
    cp = pltpu.CompilerParams()
    if "needs_layout_passes" in pltpu.CompilerParams.__dataclass_fields__:
        cp = dataclasses.replace(cp, needs_layout_passes=False)

    @pl.kernel(
        compiler_params=cp,
        out_type=jax.ShapeDtypeStruct((batch, hist, embed), table.dtype),
        mesh=mesh,
        scratch_types=[
            pltpu.VMEM((per_w,), jnp.int32),
            pltpu.VMEM((rows_blk,), jnp.int32),
            pltpu.VMEM((rows_blk,), jnp.int32),
            pltpu.VMEM((rows_blk,), jnp.int32),
            pltpu.VMEM((rows_blk,), jnp.int32),
            pltpu.VMEM((rows_blk, width), jnp.float32),
            pltpu.VMEM((rows_blk, width), jnp.float32),
            pltpu.VMEM((rows_blk, embed), jnp.float32),
            pltpu.VMEM((rows_blk, embed), jnp.float32),
            pltpu.SemaphoreType.DMA,
            pltpu.SemaphoreType.DMA,
            pltpu.SemaphoreType.DMA,
            pltpu.SemaphoreType.DMA,
        ],
    )
    def gather_kernel(
        table_hbm, idx_hbm, out_hbm,
        idx_all, half0, half1, par0, par1, rows0, rows1, out0, out1,
        gsem0, gsem1, osem0, osem1,
    ):
        wid = lax.axis_index("subcore") * _NC + lax.axis_index("core")
        base = wid * per_w
        bbase = wid * per_wb
        lanes = lax.iota(jnp.int32, 16)

        pltpu.sync_copy(idx_hbm.at[pl.ds(base, per_w)], idx_all)

        def prep(off, half, par):
            @pl.loop(0, rows_blk, step=16)
            def _(i):
                v = idx_all[pl.ds(off + i, 16)]
                half[pl.ds(i, 16)] = lax.shift_right_logical(v, 1)
                par[pl.ds(i, 16)] = lax.shift_left(lax.bitwise_and(v, 1), 6)

        def start_gather(half, rows, gsem):
            return pltpu.async_copy(table_hbm.at[half], rows, gsem)

        def select(rows, par, out):
            @plsc.parallel_loop(0, rows_blk, step=2, unroll=2)
            def _(r):
                r0 = jnp.zeros((16,), jnp.int32) + r
                r1 = r0 + 1
                p0 = plsc.load_gather(par, [r0])
                p1 = plsc.load_gather(par, [r1])
                for c in range(embed // 16):
                    x = plsc.load_gather(rows, [r0, p0 + (c * 16 + lanes)])
                    out[r, pl.ds(c * 16, 16)] = x
                for c in range(embed // 16):
                    x = plsc.load_gather(rows, [r1, p1 + (c * 16 + lanes)])
                    out[r + 1, pl.ds(c * 16, 16)] = x

        def start_out(out, chunk, osem):
            return pltpu.async_copy(
                out.reshape(_BLK, hist, embed),
                out_hbm.at[pl.ds(bbase + chunk * _BLK, _BLK)],
                osem,
            )

        prep(0, half0, par0)
        g0 = start_gather(half0, rows0, gsem0)
        prep(rows_blk, half1, par1)
        g1 = start_gather(half1, rows1, gsem1)

        @pl.loop(0, (nchunk - 2) // 2)
        def _(k):
            off = 2 * k * rows_blk
            g0.wait()
            select(rows0, par0, out0)
            prep(off + 2 * rows_blk, half0, par0)
            start_gather(half0, rows0, gsem0)
            o0 = start_out(out0, 2 * k, osem0)
            g1.wait()
            select(rows1, par1, out1)
            prep(off + 3 * rows_blk, half1, par1)
            start_gather(half1, rows1, gsem1)
            o1 = start_out(out1, 2 * k + 1, osem1)
            o0.wait()
            o1.wait()

        g0.wait()
        select(rows0, par0, out0)
        o0 = start_out(out0, nchunk - 2, osem0)
        g1.wait()
        select(rows1, par1, out1)
        o1 = start_out(out1, nchunk - 1, osem1)
        o0.wait()
        o1.wait()

    return gather_kernel(table_pairs, idx_flat)


# speedup vs baseline: 2.5890x; 1.3354x over previous
import dataclasses

import jax
import jax.numpy as jnp
from jax import lax
from jax.experimental import pallas as pl
from jax.experimental.pallas import tpu as pltpu
from jax.experimental.pallas import tpu_sc as plsc
from jax.experimental import layout as jex_layout

_NC, _NS = 2, 16
_NW = _NC * _NS
_BLK = 8


def kernel(tensor, table):
    batch, hist = tensor.shape
    vocab, embed = table.shape
    n = batch * hist
    per_wb = batch // _NW
    per_w = per_wb * hist
    rows_blk = _BLK * hist
    nchunk = per_wb // _BLK
    idx_flat = tensor.reshape(n)
    table_lin = jex_layout.with_layout_constraint(
        table, jex_layout.Layout(major_to_minor=(0, 1), tiling=())
    )

    mesh = plsc.VectorSubcoreMesh(
        core_axis_name="core", subcore_axis_name="subcore"
    )
    cp = pltpu.CompilerParams()
    if "needs_layout_passes" in pltpu.CompilerParams.__dataclass_fields__:
        cp = dataclasses.replace(cp, needs_layout_passes=False)

    @pl.kernel(
        compiler_params=cp,
        out_type=jax.ShapeDtypeStruct((batch, hist, embed), table.dtype),
        mesh=mesh,
        scratch_types=[
            pltpu.VMEM((per_w,), jnp.int32),
            pltpu.VMEM((rows_blk, embed), jnp.float32),
            pltpu.VMEM((rows_blk, embed), jnp.float32),
            pltpu.SemaphoreType.DMA,
            pltpu.SemaphoreType.DMA,
            pltpu.SemaphoreType.DMA,
            pltpu.SemaphoreType.DMA,
        ],
    )
    def gather_kernel(
        table_hbm, idx_hbm, out_hbm,
        idx_all, rows0, rows1,
        gsem0, gsem1, osem0, osem1,
    ):
        wid = lax.axis_index("subcore") * _NC + lax.axis_index("core")
        base = wid * per_w
        bbase = wid * per_wb

        pltpu.sync_copy(idx_hbm.at[pl.ds(base, per_w)], idx_all)

        def start_gather(chunk, rows, gsem):
            return pltpu.async_copy(
                table_hbm.at[idx_all.at[pl.ds(chunk * rows_blk, rows_blk)]],
                rows,
                gsem,
            )

        def start_out(rows, chunk, osem):
            return pltpu.async_copy(
                rows.reshape(_BLK, hist, embed),
                out_hbm.at[pl.ds(bbase + chunk * _BLK, _BLK)],
                osem,
            )

        g0 = start_gather(0, rows0, gsem0)
        g1 = start_gather(1, rows1, gsem1)

        @pl.loop(0, (nchunk - 2) // 2)
        def _(k):
            g0.wait()
            o0 = start_out(rows0, 2 * k, osem0)
            g1.wait()
            o1 = start_out(rows1, 2 * k + 1, osem1)
            o0.wait()
            start_gather(2 * k + 2, rows0, gsem0)
            o1.wait()
            start_gather(2 * k + 3, rows1, gsem1)

        g0.wait()
        o0 = start_out(rows0, nchunk - 2, osem0)
        g1.wait()
        o1 = start_out(rows1, nchunk - 1, osem1)
        o0.wait()
        o1.wait()

    return gather_kernel(table_lin, idx_flat)
